# baseline (speedup 1.0000x reference)
import dataclasses
import functools

import jax
import jax.numpy as jnp
from jax import lax
from jax.experimental import pallas as pl
from jax.experimental.pallas import tpu as pltpu
from jax.experimental.pallas import tpu_sc as plsc

N = 10000
E = 320000
D = 128
NC = 2
NS = 16
NW = NC * NS
K = 128
NRT = 160
NCH = NS * NRT
E2 = NCH * K
HALF = 40
PIECES = (40, 40, 40, 40)
RB = 624
REXTRA = N - NS * RB
ZCHUNKS = ((0, 128), (128, 128), (256, 128), (384, 128), (512, 112))

_sc_params = pltpu.CompilerParams()
if "needs_layout_passes" in pltpu.CompilerParams.__dataclass_fields__:
    _sc_params = dataclasses.replace(_sc_params, needs_layout_passes=False)

_mesh = plsc.VectorSubcoreMesh(
    core_axis_name="c", subcore_axis_name="s", num_cores=NC, num_subcores=NS
)


def _deg_body(w2_hbm, dst2_hbm, out_hbm, deg_sh, w_all, didx_all, zbuf_v,
              xbuf_v, ssem):
    cid = lax.axis_index("c")
    sid = lax.axis_index("s")

    @pl.when(cid == 0)
    def _core0():
        @pl.loop(0, RB // 16)
        def _zero_buf(i):
            zbuf_v[pl.ds(i * 16, 16)] = jnp.zeros((16,), jnp.float32)

        pltpu.sync_copy(zbuf_v, deg_sh.at[pl.ds(sid * RB, RB)])

        @pl.when(sid == NS - 1)
        def _zero_extra():
            pltpu.sync_copy(zbuf_v.at[pl.ds(0, REXTRA)],
                            deg_sh.at[pl.ds(NS * RB, REXTRA)])

        pltpu.sync_copy(w2_hbm.at[pl.ds(sid * NRT, NRT)], w_all)
        pltpu.sync_copy(dst2_hbm.at[pl.ds(sid * NRT, NRT)], didx_all)

        plsc.subcore_barrier()

        @pl.loop(0, NRT)
        def _fire(i):
            pltpu.async_copy(w_all.at[i], deg_sh.at[didx_all.at[i]], ssem,
                             add=True)

        @pl.loop(0, NRT)
        def _drain(i):
            pltpu.make_async_copy(w_all.at[0], deg_sh.at[didx_all.at[0]],
                                  ssem).wait()

        plsc.subcore_barrier()
        pltpu.sync_copy(deg_sh.at[pl.ds(sid * RB, RB)], zbuf_v)
        pltpu.sync_copy(zbuf_v, out_hbm.at[pl.ds(sid * RB, RB)])

        @pl.when(sid == NS - 1)
        def _copy_extra():
            pltpu.sync_copy(deg_sh.at[pl.ds(NS * RB, REXTRA)], xbuf_v)
            pltpu.sync_copy(xbuf_v, out_hbm.at[pl.ds(NS * RB, REXTRA)])


_deg_kernel = functools.partial(
    pl.kernel,
    out_type=jax.ShapeDtypeStruct((N,), jnp.float32),
    mesh=_mesh,
    scratch_types=[
        pltpu.VMEM_SHARED((N,), jnp.float32),
        pltpu.VMEM((NRT, K), jnp.float32),
        pltpu.VMEM((NRT, K), jnp.int32),
        pltpu.VMEM((RB,), jnp.float32),
        pltpu.VMEM((REXTRA,), jnp.float32),
        pltpu.SemaphoreType.DMA,
    ],
)(_deg_body)


def _msg_body(y_hbm, src2_hbm, dst2_hbm, w2_hbm, out_hbm, acc_sh, r0, r1,
              src_h, dst_h, w_h, g0, g1, s0, s1):
    cid = lax.axis_index("c")
    sid = lax.axis_index("s")
    rows = (r0, r1)
    gsem = (g0, g1)
    ssem = (s0, s1)
    def issue_gather(i, b):
        pltpu.async_copy(y_hbm.at[src_h.at[i]], rows[b], gsem[b])

    def wait_gather(b):
        pltpu.make_async_copy(y_hbm.at[src_h.at[0]], rows[b], gsem[b]).wait()

    def issue_scatter(i, b):
        pltpu.async_copy(rows[b], acc_sh.at[dst_h.at[i]], ssem[b], add=True)

    def wait_scatter(b):
        pltpu.make_async_copy(rows[0], acc_sh.at[dst_h.at[0]],
                              ssem[b]).wait()

    @pl.when(cid == 0)
    def _core0():
        @pl.loop(0, K)
        def _zero_buf(i):
            for j in range(D // 16):
                r0[i, pl.ds(j * 16, 16)] = jnp.zeros((16,), jnp.float32)

        for zoff, zsz in ZCHUNKS:
            pltpu.sync_copy(r0.at[pl.ds(0, zsz)],
                            acc_sh.at[pl.ds(sid * RB + zoff, zsz)])

        @pl.when(sid == NS - 1)
        def _zero_extra():
            pltpu.sync_copy(r0.at[pl.ds(0, REXTRA)],
                            acc_sh.at[pl.ds(NS * RB, REXTRA)])

        plsc.subcore_barrier()

        def emit_piece(base_row, nch):
            pltpu.sync_copy(src2_hbm.at[pl.ds(base_row, nch)],
                            src_h.at[pl.ds(0, nch)])
            pltpu.sync_copy(dst2_hbm.at[pl.ds(base_row, nch)],
                            dst_h.at[pl.ds(0, nch)])
            pltpu.sync_copy(w2_hbm.at[pl.ds(base_row, nch)],
                            w_h.at[pl.ds(0, nch)])

            issue_gather(0, 0)

            @pl.loop(0, nch, step=2)
            def _group(g):
                for k in range(2):
                    i = g + k
                    b = k
                    bo = 1 - k

                    @pl.when(i >= 1)
                    def _drain_prev():
                        wait_scatter(bo)

                    @pl.when(i + 1 < nch)
                    def _prefetch():
                        issue_gather(i + 1, bo)

                    wait_gather(b)

                    @pl.loop(0, K, unroll=4)
                    def _scale(e):
                        i16 = jnp.full((16,), i, jnp.int32)
                        e16 = jnp.full((16,), e, jnp.int32)
                        we = plsc.load_gather(w_h, [i16, e16])
                        for j in range(D // 16):
                            sl = (e, pl.ds(j * 16, 16))
                            rows[b][sl] = rows[b][sl] * we

                    issue_scatter(i, b)

            wait_scatter(1)

        base = sid * NRT
        for p in PIECES:
            emit_piece(base, p)
            base += p

        plsc.subcore_barrier()
        pltpu.sync_copy(acc_sh.at[pl.ds(sid * RB, RB)],
                        out_hbm.at[pl.ds(sid * RB, RB)])

        @pl.when(sid == NS - 1)
        def _copy_extra():
            pltpu.sync_copy(acc_sh.at[pl.ds(NS * RB, REXTRA)],
                            out_hbm.at[pl.ds(NS * RB, REXTRA)])


_msg_kernel = functools.partial(
    pl.kernel,
    out_type=jax.ShapeDtypeStruct((N, D), jnp.float32),
    mesh=_mesh,
    scratch_types=[
        pltpu.VMEM_SHARED((N, D), jnp.float32),
        pltpu.VMEM((K, D), jnp.float32),
        pltpu.VMEM((K, D), jnp.float32),
        pltpu.VMEM((HALF, K), jnp.int32),
        pltpu.VMEM((HALF, K), jnp.int32),
        pltpu.VMEM((HALF, K), jnp.float32),
        pltpu.SemaphoreType.DMA,
        pltpu.SemaphoreType.DMA,
        pltpu.SemaphoreType.DMA,
        pltpu.SemaphoreType.DMA,
    ],
    compiler_params=_sc_params,
)(_msg_body)


def _mm_body(x_ref, w_ref, xw_ref):
    xw_ref[...] = jnp.dot(x_ref[...], w_ref[...],
                          preferred_element_type=jnp.float32)


def _mm(x, W):
    return pl.pallas_call(
        _mm_body,
        out_shape=jax.ShapeDtypeStruct((N, D), jnp.float32),
    )(x, W)


def _scale_body(xw_ref, degp_ref, y_ref):
    deg = degp_ref[...] + 1.0
    dis = jnp.where(deg > 0, lax.rsqrt(deg), 0.0)
    y_ref[...] = xw_ref[...] * dis.reshape(N, 1)


def _scale(xw, degp):
    return pl.pallas_call(
        _scale_body,
        out_shape=jax.ShapeDtypeStruct((N, D), jnp.float32),
    )(xw, degp)


def _final_body(acc_ref, y_ref, degp_ref, b_ref, o_ref):
    deg = degp_ref[...] + 1.0
    dis = jnp.where(deg > 0, lax.rsqrt(deg), 0.0)
    agg = acc_ref[...] + y_ref[...]
    out = dis.reshape(N, 1) * agg + b_ref[...]
    m = jnp.max(out, axis=0, keepdims=True)
    z = jnp.exp(out - m)
    lse = jnp.log(jnp.sum(z, axis=0, keepdims=True))
    o_ref[...] = out - m - lse


def _final(acc, y, degp, b):
    return pl.pallas_call(
        _final_body,
        out_shape=jax.ShapeDtypeStruct((N, D), jnp.float32),
    )(acc, y, degp, b)


def kernel(x, edge_index, edge_weight, W, b):
    src = edge_index[0]
    dst = edge_index[1]
    pad = E2 - E
    src2 = jnp.pad(src, (0, pad)).reshape(NCH, K)
    dst2 = jnp.pad(dst, (0, pad)).reshape(NCH, K)
    w2 = jnp.pad(edge_weight, (0, pad)).reshape(NCH, K)
    degp = _deg_kernel(w2, dst2)
    xw = _mm(x, W)
    y = _scale(xw, degp)
    acc = _msg_kernel(y, src2, dst2, w2)
    return _final(acc, y, degp, b)


# speedup vs baseline: 1.4222x; 1.4222x over previous
import dataclasses
import functools

import jax
import jax.numpy as jnp
from jax import lax
from jax.experimental import pallas as pl
from jax.experimental.pallas import tpu as pltpu
from jax.experimental.pallas import tpu_sc as plsc

N = 10000
E = 320000
D = 128
NC = 2
NS = 16
NW = NC * NS
K = 128
NCH = 2560
E2 = NCH * K
NRT = 160
HALF = 40
CFAST = 104
CSLOW = 56
PIECES_FAST = (40, 40, 24)
PIECES_SLOW = (40, 16)
RB = 624
REXTRA = N - NS * RB
ZCHUNKS = ((0, 128), (128, 128), (256, 128), (384, 128), (512, 112))

_sc_params = pltpu.CompilerParams()
if "needs_layout_passes" in pltpu.CompilerParams.__dataclass_fields__:
    _sc_params = dataclasses.replace(_sc_params, needs_layout_passes=False)

_mesh = plsc.VectorSubcoreMesh(
    core_axis_name="c", subcore_axis_name="s", num_cores=NC, num_subcores=NS
)


def _deg_body(w2_hbm, dst2_hbm, out_hbm, deg_sh, w_all, didx_all, zbuf_v,
              xbuf_v, ssem):
    cid = lax.axis_index("c")
    sid = lax.axis_index("s")

    @pl.when(cid == 0)
    def _core0():
        @pl.loop(0, RB // 16)
        def _zero_buf(i):
            zbuf_v[pl.ds(i * 16, 16)] = jnp.zeros((16,), jnp.float32)

        pltpu.sync_copy(zbuf_v, deg_sh.at[pl.ds(sid * RB, RB)])

        @pl.when(sid == NS - 1)
        def _zero_extra():
            pltpu.sync_copy(zbuf_v.at[pl.ds(0, REXTRA)],
                            deg_sh.at[pl.ds(NS * RB, REXTRA)])

        pltpu.sync_copy(w2_hbm.at[pl.ds(sid * NRT, NRT)], w_all)
        pltpu.sync_copy(dst2_hbm.at[pl.ds(sid * NRT, NRT)], didx_all)

        plsc.subcore_barrier()

        @pl.loop(0, NRT)
        def _fire(i):
            pltpu.async_copy(w_all.at[i], deg_sh.at[didx_all.at[i]], ssem,
                             add=True)

        @pl.loop(0, NRT)
        def _drain(i):
            pltpu.make_async_copy(w_all.at[0], deg_sh.at[didx_all.at[0]],
                                  ssem).wait()

        plsc.subcore_barrier()
        pltpu.sync_copy(deg_sh.at[pl.ds(sid * RB, RB)], zbuf_v)
        pltpu.sync_copy(zbuf_v, out_hbm.at[pl.ds(sid * RB, RB)])

        @pl.when(sid == NS - 1)
        def _copy_extra():
            pltpu.sync_copy(deg_sh.at[pl.ds(NS * RB, REXTRA)], xbuf_v)
            pltpu.sync_copy(xbuf_v, out_hbm.at[pl.ds(NS * RB, REXTRA)])


_deg_kernel = functools.partial(
    pl.kernel,
    out_type=jax.ShapeDtypeStruct((N,), jnp.float32),
    mesh=_mesh,
    scratch_types=[
        pltpu.VMEM_SHARED((N,), jnp.float32),
        pltpu.VMEM((NRT, K), jnp.float32),
        pltpu.VMEM((NRT, K), jnp.int32),
        pltpu.VMEM((RB,), jnp.float32),
        pltpu.VMEM((REXTRA,), jnp.float32),
        pltpu.SemaphoreType.DMA,
    ],
)(_deg_body)


def _msg_body(y_hbm, src2_hbm, dst2_hbm, w2_hbm, out_hbm, acc_sh, r0, r1,
              src_h, dst_h, w_h, g0, g1, s0, s1):
    cid = lax.axis_index("c")
    sid = lax.axis_index("s")
    rows = (r0, r1)
    gsem = (g0, g1)
    ssem = (s0, s1)
    def issue_gather(i, b):
        pltpu.async_copy(y_hbm.at[src_h.at[i]], rows[b], gsem[b])

    def wait_gather(b):
        pltpu.make_async_copy(y_hbm.at[src_h.at[0]], rows[b], gsem[b]).wait()

    def issue_scatter(i, b):
        pltpu.async_copy(rows[b], acc_sh.at[dst_h.at[i]], ssem[b], add=True)

    def wait_scatter(b):
        pltpu.make_async_copy(rows[0], acc_sh.at[dst_h.at[0]],
                              ssem[b]).wait()

    @pl.loop(0, K)
    def _zero_buf(i):
        for j in range(D // 16):
            r0[i, pl.ds(j * 16, 16)] = jnp.zeros((16,), jnp.float32)

    for zoff, zsz in ZCHUNKS:
        pltpu.sync_copy(r0.at[pl.ds(0, zsz)],
                        acc_sh.at[pl.ds(sid * RB + zoff, zsz)])

    @pl.when(sid == NS - 1)
    def _zero_extra():
        pltpu.sync_copy(r0.at[pl.ds(0, REXTRA)],
                        acc_sh.at[pl.ds(NS * RB, REXTRA)])

    plsc.subcore_barrier()

    def emit_piece(base_row, nch):
        pltpu.sync_copy(src2_hbm.at[pl.ds(base_row, nch)],
                        src_h.at[pl.ds(0, nch)])
        pltpu.sync_copy(dst2_hbm.at[pl.ds(base_row, nch)],
                        dst_h.at[pl.ds(0, nch)])
        pltpu.sync_copy(w2_hbm.at[pl.ds(base_row, nch)],
                        w_h.at[pl.ds(0, nch)])

        issue_gather(0, 0)

        @pl.loop(0, nch, step=2)
        def _group(g):
            for k in range(2):
                i = g + k
                b = k
                bo = 1 - k

                @pl.when(i >= 1)
                def _drain_prev():
                    wait_scatter(bo)

                @pl.when(i + 1 < nch)
                def _prefetch():
                    issue_gather(i + 1, bo)

                wait_gather(b)

                @pl.loop(0, K, unroll=4)
                def _scale(e):
                    i16 = jnp.full((16,), i, jnp.int32)
                    e16 = jnp.full((16,), e, jnp.int32)
                    we = plsc.load_gather(w_h, [i16, e16])
                    for j in range(D // 16):
                        sl = (e, pl.ds(j * 16, 16))
                        rows[b][sl] = rows[b][sl] * we

                issue_scatter(i, b)

        wait_scatter(1)

    @pl.when(cid == 0)
    def _fast_core():
        base = sid * CFAST
        for p in PIECES_FAST:
            emit_piece(base, p)
            base += p

    @pl.when(cid != 0)
    def _slow_core():
        base = NS * CFAST + sid * CSLOW
        for p in PIECES_SLOW:
            emit_piece(base, p)
            base += p

    plsc.subcore_barrier()
    pltpu.sync_copy(acc_sh.at[pl.ds(sid * RB, RB)],
                    out_hbm.at[cid, pl.ds(sid * RB, RB)])

    @pl.when(sid == NS - 1)
    def _copy_extra():
        pltpu.sync_copy(acc_sh.at[pl.ds(NS * RB, REXTRA)],
                        out_hbm.at[cid, pl.ds(NS * RB, REXTRA)])


_msg_kernel = functools.partial(
    pl.kernel,
    out_type=jax.ShapeDtypeStruct((NC, N, D), jnp.float32),
    mesh=_mesh,
    scratch_types=[
        pltpu.VMEM_SHARED((N, D), jnp.float32),
        pltpu.VMEM((K, D), jnp.float32),
        pltpu.VMEM((K, D), jnp.float32),
        pltpu.VMEM((HALF, K), jnp.int32),
        pltpu.VMEM((HALF, K), jnp.int32),
        pltpu.VMEM((HALF, K), jnp.float32),
        pltpu.SemaphoreType.DMA,
        pltpu.SemaphoreType.DMA,
        pltpu.SemaphoreType.DMA,
        pltpu.SemaphoreType.DMA,
    ],
    compiler_params=_sc_params,
)(_msg_body)


def _mm_body(x_ref, w_ref, xw_ref):
    xw_ref[...] = jnp.dot(x_ref[...], w_ref[...],
                          preferred_element_type=jnp.float32)


def _mm(x, W):
    return pl.pallas_call(
        _mm_body,
        out_shape=jax.ShapeDtypeStruct((N, D), jnp.float32),
    )(x, W)


def _scale_body(xw_ref, degp_ref, y_ref):
    deg = degp_ref[...] + 1.0
    dis = jnp.where(deg > 0, lax.rsqrt(deg), 0.0)
    y_ref[...] = xw_ref[...] * dis.reshape(N, 1)


def _scale(xw, degp):
    return pl.pallas_call(
        _scale_body,
        out_shape=jax.ShapeDtypeStruct((N, D), jnp.float32),
    )(xw, degp)


def _final_body(acc_ref, y_ref, degp_ref, b_ref, o_ref):
    deg = degp_ref[...] + 1.0
    dis = jnp.where(deg > 0, lax.rsqrt(deg), 0.0)
    agg = acc_ref[0] + acc_ref[1] + y_ref[...]
    out = dis.reshape(N, 1) * agg + b_ref[...]
    m = jnp.max(out, axis=0, keepdims=True)
    z = jnp.exp(out - m)
    lse = jnp.log(jnp.sum(z, axis=0, keepdims=True))
    o_ref[...] = out - m - lse


def _final(acc, y, degp, b):
    return pl.pallas_call(
        _final_body,
        out_shape=jax.ShapeDtypeStruct((N, D), jnp.float32),
    )(acc, y, degp, b)


def kernel(x, edge_index, edge_weight, W, b):
    src = edge_index[0]
    dst = edge_index[1]
    pad = E2 - E
    src2 = jnp.pad(src, (0, pad)).reshape(NCH, K)
    dst2 = jnp.pad(dst, (0, pad)).reshape(NCH, K)
    w2 = jnp.pad(edge_weight, (0, pad)).reshape(NCH, K)
    degp = _deg_kernel(w2, dst2)
    xw = _mm(x, W)
    y = _scale(xw, degp)
    acc = _msg_kernel(y, src2, dst2, w2)
    return _final(acc, y, degp, b)


# speedup vs baseline: 1.5080x; 1.0603x over previous
import dataclasses
import functools

import jax
import jax.numpy as jnp
from jax import lax
from jax.experimental import pallas as pl
from jax.experimental.pallas import tpu as pltpu
from jax.experimental.pallas import tpu_sc as plsc

N = 10000
E = 320000
D = 128
NC = 2
NS = 16
NW = NC * NS
K = 128
NCH = 2560
E2 = NCH * K
NRT = 80
HALF = 40
CFAST = 104
CSLOW = 56
PIECES_FAST = (40, 40, 24)
PIECES_SLOW = (40, 16)
RB = 624
REXTRA = N - NS * RB
ZCHUNKS = ((0, 128), (128, 128), (256, 128), (384, 128), (512, 112))

_sc_params = pltpu.CompilerParams()
if "needs_layout_passes" in pltpu.CompilerParams.__dataclass_fields__:
    _sc_params = dataclasses.replace(_sc_params, needs_layout_passes=False)

_mesh = plsc.VectorSubcoreMesh(
    core_axis_name="c", subcore_axis_name="s", num_cores=NC, num_subcores=NS
)


def _deg_body(w2_hbm, dst2_hbm, out_hbm, deg_sh, w_all, didx_all, zbuf_v,
              xbuf_v, ssem):
    cid = lax.axis_index("c")
    sid = lax.axis_index("s")
    wid = cid * NS + sid

    @pl.loop(0, RB // 16)
    def _zero_buf(i):
        zbuf_v[pl.ds(i * 16, 16)] = jnp.zeros((16,), jnp.float32)

    pltpu.sync_copy(zbuf_v, deg_sh.at[pl.ds(sid * RB, RB)])

    @pl.when(sid == NS - 1)
    def _zero_extra():
        pltpu.sync_copy(zbuf_v.at[pl.ds(0, REXTRA)],
                        deg_sh.at[pl.ds(NS * RB, REXTRA)])

    pltpu.sync_copy(w2_hbm.at[pl.ds(wid * NRT, NRT)], w_all)
    pltpu.sync_copy(dst2_hbm.at[pl.ds(wid * NRT, NRT)], didx_all)

    plsc.subcore_barrier()

    @pl.loop(0, NRT)
    def _fire(i):
        pltpu.async_copy(w_all.at[i], deg_sh.at[didx_all.at[i]], ssem,
                         add=True)

    @pl.loop(0, NRT)
    def _drain(i):
        pltpu.make_async_copy(w_all.at[0], deg_sh.at[didx_all.at[0]],
                              ssem).wait()

    plsc.subcore_barrier()
    pltpu.sync_copy(deg_sh.at[pl.ds(sid * RB, RB)], zbuf_v)
    pltpu.sync_copy(zbuf_v, out_hbm.at[pl.ds(cid * N + sid * RB, RB)])

    @pl.when(sid == NS - 1)
    def _copy_extra():
        pltpu.sync_copy(deg_sh.at[pl.ds(NS * RB, REXTRA)], xbuf_v)
        pltpu.sync_copy(xbuf_v, out_hbm.at[pl.ds(cid * N + NS * RB, REXTRA)])


_deg_kernel = functools.partial(
    pl.kernel,
    out_type=jax.ShapeDtypeStruct((NC * N,), jnp.float32),
    mesh=_mesh,
    scratch_types=[
        pltpu.VMEM_SHARED((N,), jnp.float32),
        pltpu.VMEM((NRT, K), jnp.float32),
        pltpu.VMEM((NRT, K), jnp.int32),
        pltpu.VMEM((RB,), jnp.float32),
        pltpu.VMEM((REXTRA,), jnp.float32),
        pltpu.SemaphoreType.DMA,
    ],
)(_deg_body)


def _msg_body(y_hbm, src2_hbm, dst2_hbm, w2_hbm, out_hbm, acc_sh, r0, r1,
              src_h, dst_h, w_h, g0, g1, s0, s1):
    cid = lax.axis_index("c")
    sid = lax.axis_index("s")
    rows = (r0, r1)
    gsem = (g0, g1)
    ssem = (s0, s1)
    def issue_gather(i, b):
        pltpu.async_copy(y_hbm.at[src_h.at[i]], rows[b], gsem[b])

    def wait_gather(b):
        pltpu.make_async_copy(y_hbm.at[src_h.at[0]], rows[b], gsem[b]).wait()

    def issue_scatter(i, b):
        pltpu.async_copy(rows[b], acc_sh.at[dst_h.at[i]], ssem[b], add=True)

    def wait_scatter(b):
        pltpu.make_async_copy(rows[0], acc_sh.at[dst_h.at[0]],
                              ssem[b]).wait()

    @pl.loop(0, K)
    def _zero_buf(i):
        for j in range(D // 16):
            r0[i, pl.ds(j * 16, 16)] = jnp.zeros((16,), jnp.float32)

    for zoff, zsz in ZCHUNKS:
        pltpu.sync_copy(r0.at[pl.ds(0, zsz)],
                        acc_sh.at[pl.ds(sid * RB + zoff, zsz)])

    @pl.when(sid == NS - 1)
    def _zero_extra():
        pltpu.sync_copy(r0.at[pl.ds(0, REXTRA)],
                        acc_sh.at[pl.ds(NS * RB, REXTRA)])

    plsc.subcore_barrier()

    def emit_piece(base_row, nch):
        pltpu.sync_copy(src2_hbm.at[pl.ds(base_row, nch)],
                        src_h.at[pl.ds(0, nch)])
        pltpu.sync_copy(dst2_hbm.at[pl.ds(base_row, nch)],
                        dst_h.at[pl.ds(0, nch)])
        pltpu.sync_copy(w2_hbm.at[pl.ds(base_row, nch)],
                        w_h.at[pl.ds(0, nch)])

        issue_gather(0, 0)

        @pl.loop(0, nch, step=2)
        def _group(g):
            for k in range(2):
                i = g + k
                b = k
                bo = 1 - k

                @pl.when(i >= 1)
                def _drain_prev():
                    wait_scatter(bo)

                @pl.when(i + 1 < nch)
                def _prefetch():
                    issue_gather(i + 1, bo)

                wait_gather(b)

                @pl.loop(0, K, unroll=4)
                def _scale(e):
                    i16 = jnp.full((16,), i, jnp.int32)
                    e16 = jnp.full((16,), e, jnp.int32)
                    we = plsc.load_gather(w_h, [i16, e16])
                    for j in range(D // 16):
                        sl = (e, pl.ds(j * 16, 16))
                        rows[b][sl] = rows[b][sl] * we

                issue_scatter(i, b)

        wait_scatter(1)

    @pl.when(cid == 0)
    def _fast_core():
        base = sid * CFAST
        for p in PIECES_FAST:
            emit_piece(base, p)
            base += p

    @pl.when(cid != 0)
    def _slow_core():
        base = NS * CFAST + sid * CSLOW
        for p in PIECES_SLOW:
            emit_piece(base, p)
            base += p

    plsc.subcore_barrier()
    pltpu.sync_copy(acc_sh.at[pl.ds(sid * RB, RB)],
                    out_hbm.at[cid, pl.ds(sid * RB, RB)])

    @pl.when(sid == NS - 1)
    def _copy_extra():
        pltpu.sync_copy(acc_sh.at[pl.ds(NS * RB, REXTRA)],
                        out_hbm.at[cid, pl.ds(NS * RB, REXTRA)])


_msg_kernel = functools.partial(
    pl.kernel,
    out_type=jax.ShapeDtypeStruct((NC, N, D), jnp.float32),
    mesh=_mesh,
    scratch_types=[
        pltpu.VMEM_SHARED((N, D), jnp.float32),
        pltpu.VMEM((K, D), jnp.float32),
        pltpu.VMEM((K, D), jnp.float32),
        pltpu.VMEM((HALF, K), jnp.int32),
        pltpu.VMEM((HALF, K), jnp.int32),
        pltpu.VMEM((HALF, K), jnp.float32),
        pltpu.SemaphoreType.DMA,
        pltpu.SemaphoreType.DMA,
        pltpu.SemaphoreType.DMA,
        pltpu.SemaphoreType.DMA,
    ],
    compiler_params=_sc_params,
)(_msg_body)


def _mm_body(x_ref, w_ref, xw_ref):
    xw_ref[...] = jnp.dot(x_ref[...], w_ref[...],
                          preferred_element_type=jnp.float32)


def _mm(x, W):
    return pl.pallas_call(
        _mm_body,
        out_shape=jax.ShapeDtypeStruct((N, D), jnp.float32),
    )(x, W)


def _scale_body(xw_ref, degp_ref, y_ref):
    deg = degp_ref[0] + degp_ref[1] + 1.0
    dis = jnp.where(deg > 0, lax.rsqrt(deg), 0.0)
    y_ref[...] = xw_ref[...] * dis.reshape(N, 1)


def _scale(xw, degp):
    return pl.pallas_call(
        _scale_body,
        out_shape=jax.ShapeDtypeStruct((N, D), jnp.float32),
    )(xw, degp)


def _final_body(acc_ref, y_ref, degp_ref, b_ref, o_ref):
    deg = degp_ref[0] + degp_ref[1] + 1.0
    dis = jnp.where(deg > 0, lax.rsqrt(deg), 0.0)
    agg = acc_ref[0] + acc_ref[1] + y_ref[...]
    out = dis.reshape(N, 1) * agg + b_ref[...]
    m = jnp.max(out, axis=0, keepdims=True)
    z = jnp.exp(out - m)
    lse = jnp.log(jnp.sum(z, axis=0, keepdims=True))
    o_ref[...] = out - m - lse


def _final(acc, y, degp, b):
    return pl.pallas_call(
        _final_body,
        out_shape=jax.ShapeDtypeStruct((N, D), jnp.float32),
    )(acc, y, degp, b)


def kernel(x, edge_index, edge_weight, W, b):
    src = edge_index[0]
    dst = edge_index[1]
    pad = E2 - E
    src2 = jnp.pad(src, (0, pad)).reshape(NCH, K)
    dst2 = jnp.pad(dst, (0, pad)).reshape(NCH, K)
    w2 = jnp.pad(edge_weight, (0, pad)).reshape(NCH, K)
    degp = _deg_kernel(w2, dst2).reshape(NC, N)
    xw = _mm(x, W)
    y = _scale(xw, degp)
    acc = _msg_kernel(y, src2, dst2, w2)
    return _final(acc, y, degp, b)


# speedup vs baseline: 1.5163x; 1.0055x over previous
import dataclasses
import functools

import jax
import jax.numpy as jnp
from jax import lax
from jax.experimental import pallas as pl
from jax.experimental.pallas import tpu as pltpu
from jax.experimental.pallas import tpu_sc as plsc

N = 10000
E = 320000
D = 128
NC = 2
NS = 16
NW = NC * NS
K = 128
NCH = 2560
E2 = NCH * K
NRT = 80
HALF = 40
CFAST = 112
CSLOW = 48
PIECES_FAST = (40, 40, 32)
PIECES_SLOW = (40, 8)
RB = 624
REXTRA = N - NS * RB
ZCHUNKS = ((0, 128), (128, 128), (256, 128), (384, 128), (512, 112))

_sc_params = pltpu.CompilerParams()
if "needs_layout_passes" in pltpu.CompilerParams.__dataclass_fields__:
    _sc_params = dataclasses.replace(_sc_params, needs_layout_passes=False)

_mesh = plsc.VectorSubcoreMesh(
    core_axis_name="c", subcore_axis_name="s", num_cores=NC, num_subcores=NS
)


def _deg_body(w2_hbm, dst2_hbm, out_hbm, deg_sh, w_all, didx_all, zbuf_v,
              xbuf_v, ssem):
    cid = lax.axis_index("c")
    sid = lax.axis_index("s")
    wid = cid * NS + sid

    @pl.loop(0, RB // 16)
    def _zero_buf(i):
        zbuf_v[pl.ds(i * 16, 16)] = jnp.zeros((16,), jnp.float32)

    pltpu.sync_copy(zbuf_v, deg_sh.at[pl.ds(sid * RB, RB)])

    @pl.when(sid == NS - 1)
    def _zero_extra():
        pltpu.sync_copy(zbuf_v.at[pl.ds(0, REXTRA)],
                        deg_sh.at[pl.ds(NS * RB, REXTRA)])

    pltpu.sync_copy(w2_hbm.at[pl.ds(wid * NRT, NRT)], w_all)
    pltpu.sync_copy(dst2_hbm.at[pl.ds(wid * NRT, NRT)], didx_all)

    plsc.subcore_barrier()

    @pl.loop(0, NRT)
    def _fire(i):
        pltpu.async_copy(w_all.at[i], deg_sh.at[didx_all.at[i]], ssem,
                         add=True)

    @pl.loop(0, NRT)
    def _drain(i):
        pltpu.make_async_copy(w_all.at[0], deg_sh.at[didx_all.at[0]],
                              ssem).wait()

    plsc.subcore_barrier()
    pltpu.sync_copy(deg_sh.at[pl.ds(sid * RB, RB)], zbuf_v)
    pltpu.sync_copy(zbuf_v, out_hbm.at[pl.ds(cid * N + sid * RB, RB)])

    @pl.when(sid == NS - 1)
    def _copy_extra():
        pltpu.sync_copy(deg_sh.at[pl.ds(NS * RB, REXTRA)], xbuf_v)
        pltpu.sync_copy(xbuf_v, out_hbm.at[pl.ds(cid * N + NS * RB, REXTRA)])


_deg_kernel = functools.partial(
    pl.kernel,
    out_type=jax.ShapeDtypeStruct((NC * N,), jnp.float32),
    mesh=_mesh,
    scratch_types=[
        pltpu.VMEM_SHARED((N,), jnp.float32),
        pltpu.VMEM((NRT, K), jnp.float32),
        pltpu.VMEM((NRT, K), jnp.int32),
        pltpu.VMEM((RB,), jnp.float32),
        pltpu.VMEM((REXTRA,), jnp.float32),
        pltpu.SemaphoreType.DMA,
    ],
)(_deg_body)


def _msg_body(y_hbm, src2_hbm, dst2_hbm, w2_hbm, out_hbm, acc_sh, r0, r1,
              src_h, dst_h, w_h, g0, g1, s0, s1):
    cid = lax.axis_index("c")
    sid = lax.axis_index("s")
    rows = (r0, r1)
    gsem = (g0, g1)
    ssem = (s0, s1)
    def issue_gather(i, b):
        pltpu.async_copy(y_hbm.at[src_h.at[i]], rows[b], gsem[b])

    def wait_gather(b):
        pltpu.make_async_copy(y_hbm.at[src_h.at[0]], rows[b], gsem[b]).wait()

    def issue_scatter(i, b):
        pltpu.async_copy(rows[b], acc_sh.at[dst_h.at[i]], ssem[b], add=True)

    def wait_scatter(b):
        pltpu.make_async_copy(rows[0], acc_sh.at[dst_h.at[0]],
                              ssem[b]).wait()

    @pl.loop(0, K)
    def _zero_buf(i):
        for j in range(D // 16):
            r0[i, pl.ds(j * 16, 16)] = jnp.zeros((16,), jnp.float32)

    for zoff, zsz in ZCHUNKS:
        pltpu.sync_copy(r0.at[pl.ds(0, zsz)],
                        acc_sh.at[pl.ds(sid * RB + zoff, zsz)])

    @pl.when(sid == NS - 1)
    def _zero_extra():
        pltpu.sync_copy(r0.at[pl.ds(0, REXTRA)],
                        acc_sh.at[pl.ds(NS * RB, REXTRA)])

    plsc.subcore_barrier()

    def emit_piece(base_row, nch):
        pltpu.sync_copy(src2_hbm.at[pl.ds(base_row, nch)],
                        src_h.at[pl.ds(0, nch)])
        pltpu.sync_copy(dst2_hbm.at[pl.ds(base_row, nch)],
                        dst_h.at[pl.ds(0, nch)])
        pltpu.sync_copy(w2_hbm.at[pl.ds(base_row, nch)],
                        w_h.at[pl.ds(0, nch)])

        issue_gather(0, 0)

        @pl.loop(0, nch, step=2)
        def _group(g):
            for k in range(2):
                i = g + k
                b = k
                bo = 1 - k

                @pl.when(i >= 1)
                def _drain_prev():
                    wait_scatter(bo)

                @pl.when(i + 1 < nch)
                def _prefetch():
                    issue_gather(i + 1, bo)

                wait_gather(b)

                @pl.loop(0, K, unroll=4)
                def _scale(e):
                    i16 = jnp.full((16,), i, jnp.int32)
                    e16 = jnp.full((16,), e, jnp.int32)
                    we = plsc.load_gather(w_h, [i16, e16])
                    for j in range(D // 16):
                        sl = (e, pl.ds(j * 16, 16))
                        rows[b][sl] = rows[b][sl] * we

                issue_scatter(i, b)

        wait_scatter(1)

    @pl.when(cid == 0)
    def _fast_core():
        base = sid * CFAST
        for p in PIECES_FAST:
            emit_piece(base, p)
            base += p

    @pl.when(cid != 0)
    def _slow_core():
        base = NS * CFAST + sid * CSLOW
        for p in PIECES_SLOW:
            emit_piece(base, p)
            base += p

    plsc.subcore_barrier()
    pltpu.sync_copy(acc_sh.at[pl.ds(sid * RB, RB)],
                    out_hbm.at[cid, pl.ds(sid * RB, RB)])

    @pl.when(sid == NS - 1)
    def _copy_extra():
        pltpu.sync_copy(acc_sh.at[pl.ds(NS * RB, REXTRA)],
                        out_hbm.at[cid, pl.ds(NS * RB, REXTRA)])


_msg_kernel = functools.partial(
    pl.kernel,
    out_type=jax.ShapeDtypeStruct((NC, N, D), jnp.float32),
    mesh=_mesh,
    scratch_types=[
        pltpu.VMEM_SHARED((N, D), jnp.float32),
        pltpu.VMEM((K, D), jnp.float32),
        pltpu.VMEM((K, D), jnp.float32),
        pltpu.VMEM((HALF, K), jnp.int32),
        pltpu.VMEM((HALF, K), jnp.int32),
        pltpu.VMEM((HALF, K), jnp.float32),
        pltpu.SemaphoreType.DMA,
        pltpu.SemaphoreType.DMA,
        pltpu.SemaphoreType.DMA,
        pltpu.SemaphoreType.DMA,
    ],
    compiler_params=_sc_params,
)(_msg_body)


def _mm_body(x_ref, w_ref, xw_ref):
    xw_ref[...] = jnp.dot(x_ref[...], w_ref[...],
                          preferred_element_type=jnp.float32)


def _mm(x, W):
    return pl.pallas_call(
        _mm_body,
        out_shape=jax.ShapeDtypeStruct((N, D), jnp.float32),
    )(x, W)


def _scale_body(xw_ref, degp_ref, y_ref):
    deg = degp_ref[0] + degp_ref[1] + 1.0
    dis = jnp.where(deg > 0, lax.rsqrt(deg), 0.0)
    y_ref[...] = xw_ref[...] * dis.reshape(N, 1)


def _scale(xw, degp):
    return pl.pallas_call(
        _scale_body,
        out_shape=jax.ShapeDtypeStruct((N, D), jnp.float32),
    )(xw, degp)


def _final_body(acc_ref, y_ref, degp_ref, b_ref, o_ref):
    deg = degp_ref[0] + degp_ref[1] + 1.0
    dis = jnp.where(deg > 0, lax.rsqrt(deg), 0.0)
    agg = acc_ref[0] + acc_ref[1] + y_ref[...]
    out = dis.reshape(N, 1) * agg + b_ref[...]
    m = jnp.max(out, axis=0, keepdims=True)
    z = jnp.exp(out - m)
    lse = jnp.log(jnp.sum(z, axis=0, keepdims=True))
    o_ref[...] = out - m - lse


def _final(acc, y, degp, b):
    return pl.pallas_call(
        _final_body,
        out_shape=jax.ShapeDtypeStruct((N, D), jnp.float32),
    )(acc, y, degp, b)


def kernel(x, edge_index, edge_weight, W, b):
    src = edge_index[0]
    dst = edge_index[1]
    pad = E2 - E
    src2 = jnp.pad(src, (0, pad)).reshape(NCH, K)
    dst2 = jnp.pad(dst, (0, pad)).reshape(NCH, K)
    w2 = jnp.pad(edge_weight, (0, pad)).reshape(NCH, K)
    degp = _deg_kernel(w2, dst2).reshape(NC, N)
    xw = _mm(x, W)
    y = _scale(xw, degp)
    acc = _msg_kernel(y, src2, dst2, w2)
    return _final(acc, y, degp, b)


# speedup vs baseline: 1.5439x; 1.0182x over previous
import dataclasses
import functools

import jax
import jax.numpy as jnp
from jax import lax
from jax.experimental import pallas as pl
from jax.experimental.pallas import tpu as pltpu
from jax.experimental.pallas import tpu_sc as plsc

N = 10000
E = 320000
D = 128
NC = 2
NS = 16
NW = NC * NS
K = 128
NCH = 2560
E2 = NCH * K
NRT = 80
HALF = 40
CFAST = 120
CSLOW = 40
PIECES_FAST = (40, 40, 40)
PIECES_SLOW = (40,)
RB = 624
REXTRA = N - NS * RB
ZCHUNKS = ((0, 128), (128, 128), (256, 128), (384, 128), (512, 112))

_sc_params = pltpu.CompilerParams()
if "needs_layout_passes" in pltpu.CompilerParams.__dataclass_fields__:
    _sc_params = dataclasses.replace(_sc_params, needs_layout_passes=False)

_mesh = plsc.VectorSubcoreMesh(
    core_axis_name="c", subcore_axis_name="s", num_cores=NC, num_subcores=NS
)


def _deg_body(w2_hbm, dst2_hbm, out_hbm, deg_sh, w_all, didx_all, zbuf_v,
              xbuf_v, ssem):
    cid = lax.axis_index("c")
    sid = lax.axis_index("s")
    wid = cid * NS + sid

    @pl.loop(0, RB // 16)
    def _zero_buf(i):
        zbuf_v[pl.ds(i * 16, 16)] = jnp.zeros((16,), jnp.float32)

    pltpu.sync_copy(zbuf_v, deg_sh.at[pl.ds(sid * RB, RB)])

    @pl.when(sid == NS - 1)
    def _zero_extra():
        pltpu.sync_copy(zbuf_v.at[pl.ds(0, REXTRA)],
                        deg_sh.at[pl.ds(NS * RB, REXTRA)])

    pltpu.sync_copy(w2_hbm.at[pl.ds(wid * NRT, NRT)], w_all)
    pltpu.sync_copy(dst2_hbm.at[pl.ds(wid * NRT, NRT)], didx_all)

    plsc.subcore_barrier()

    @pl.loop(0, NRT)
    def _fire(i):
        pltpu.async_copy(w_all.at[i], deg_sh.at[didx_all.at[i]], ssem,
                         add=True)

    @pl.loop(0, NRT)
    def _drain(i):
        pltpu.make_async_copy(w_all.at[0], deg_sh.at[didx_all.at[0]],
                              ssem).wait()

    plsc.subcore_barrier()
    pltpu.sync_copy(deg_sh.at[pl.ds(sid * RB, RB)], zbuf_v)
    pltpu.sync_copy(zbuf_v, out_hbm.at[pl.ds(cid * N + sid * RB, RB)])

    @pl.when(sid == NS - 1)
    def _copy_extra():
        pltpu.sync_copy(deg_sh.at[pl.ds(NS * RB, REXTRA)], xbuf_v)
        pltpu.sync_copy(xbuf_v, out_hbm.at[pl.ds(cid * N + NS * RB, REXTRA)])


_deg_kernel = functools.partial(
    pl.kernel,
    out_type=jax.ShapeDtypeStruct((NC * N,), jnp.float32),
    mesh=_mesh,
    scratch_types=[
        pltpu.VMEM_SHARED((N,), jnp.float32),
        pltpu.VMEM((NRT, K), jnp.float32),
        pltpu.VMEM((NRT, K), jnp.int32),
        pltpu.VMEM((RB,), jnp.float32),
        pltpu.VMEM((REXTRA,), jnp.float32),
        pltpu.SemaphoreType.DMA,
    ],
)(_deg_body)


def _msg_body(y_hbm, src2_hbm, dst2_hbm, w2_hbm, out_hbm, acc_sh, r0, r1,
              src_h, dst_h, w_h, g0, g1, s0, s1):
    cid = lax.axis_index("c")
    sid = lax.axis_index("s")
    rows = (r0, r1)
    gsem = (g0, g1)
    ssem = (s0, s1)
    def issue_gather(i, b):
        pltpu.async_copy(y_hbm.at[src_h.at[i]], rows[b], gsem[b])

    def wait_gather(b):
        pltpu.make_async_copy(y_hbm.at[src_h.at[0]], rows[b], gsem[b]).wait()

    def issue_scatter(i, b):
        pltpu.async_copy(rows[b], acc_sh.at[dst_h.at[i]], ssem[b], add=True)

    def wait_scatter(b):
        pltpu.make_async_copy(rows[0], acc_sh.at[dst_h.at[0]],
                              ssem[b]).wait()

    @pl.loop(0, K)
    def _zero_buf(i):
        for j in range(D // 16):
            r0[i, pl.ds(j * 16, 16)] = jnp.zeros((16,), jnp.float32)

    for zoff, zsz in ZCHUNKS:
        pltpu.sync_copy(r0.at[pl.ds(0, zsz)],
                        acc_sh.at[pl.ds(sid * RB + zoff, zsz)])

    @pl.when(sid == NS - 1)
    def _zero_extra():
        pltpu.sync_copy(r0.at[pl.ds(0, REXTRA)],
                        acc_sh.at[pl.ds(NS * RB, REXTRA)])

    plsc.subcore_barrier()

    def emit_piece(base_row, nch):
        pltpu.sync_copy(src2_hbm.at[pl.ds(base_row, nch)],
                        src_h.at[pl.ds(0, nch)])
        pltpu.sync_copy(dst2_hbm.at[pl.ds(base_row, nch)],
                        dst_h.at[pl.ds(0, nch)])
        pltpu.sync_copy(w2_hbm.at[pl.ds(base_row, nch)],
                        w_h.at[pl.ds(0, nch)])

        issue_gather(0, 0)

        @pl.loop(0, nch, step=2)
        def _group(g):
            for k in range(2):
                i = g + k
                b = k
                bo = 1 - k

                @pl.when(i >= 1)
                def _drain_prev():
                    wait_scatter(bo)

                @pl.when(i + 1 < nch)
                def _prefetch():
                    issue_gather(i + 1, bo)

                wait_gather(b)

                @pl.loop(0, K, unroll=4)
                def _scale(e):
                    i16 = jnp.full((16,), i, jnp.int32)
                    e16 = jnp.full((16,), e, jnp.int32)
                    we = plsc.load_gather(w_h, [i16, e16])
                    for j in range(D // 16):
                        sl = (e, pl.ds(j * 16, 16))
                        rows[b][sl] = rows[b][sl] * we

                issue_scatter(i, b)

        wait_scatter(1)

    @pl.when(cid == 0)
    def _fast_core():
        base = sid * CFAST
        for p in PIECES_FAST:
            emit_piece(base, p)
            base += p

    @pl.when(cid != 0)
    def _slow_core():
        base = NS * CFAST + sid * CSLOW
        for p in PIECES_SLOW:
            emit_piece(base, p)
            base += p

    plsc.subcore_barrier()
    pltpu.sync_copy(acc_sh.at[pl.ds(sid * RB, RB)],
                    out_hbm.at[cid, pl.ds(sid * RB, RB)])

    @pl.when(sid == NS - 1)
    def _copy_extra():
        pltpu.sync_copy(acc_sh.at[pl.ds(NS * RB, REXTRA)],
                        out_hbm.at[cid, pl.ds(NS * RB, REXTRA)])


_msg_kernel = functools.partial(
    pl.kernel,
    out_type=jax.ShapeDtypeStruct((NC, N, D), jnp.float32),
    mesh=_mesh,
    scratch_types=[
        pltpu.VMEM_SHARED((N, D), jnp.float32),
        pltpu.VMEM((K, D), jnp.float32),
        pltpu.VMEM((K, D), jnp.float32),
        pltpu.VMEM((HALF, K), jnp.int32),
        pltpu.VMEM((HALF, K), jnp.int32),
        pltpu.VMEM((HALF, K), jnp.float32),
        pltpu.SemaphoreType.DMA,
        pltpu.SemaphoreType.DMA,
        pltpu.SemaphoreType.DMA,
        pltpu.SemaphoreType.DMA,
    ],
    compiler_params=_sc_params,
)(_msg_body)


def _mm_body(x_ref, w_ref, xw_ref):
    xw_ref[...] = jnp.dot(x_ref[...], w_ref[...],
                          preferred_element_type=jnp.float32)


def _mm(x, W):
    return pl.pallas_call(
        _mm_body,
        out_shape=jax.ShapeDtypeStruct((N, D), jnp.float32),
    )(x, W)


def _scale_body(xw_ref, degp_ref, y_ref):
    deg = degp_ref[0] + degp_ref[1] + 1.0
    dis = jnp.where(deg > 0, lax.rsqrt(deg), 0.0)
    y_ref[...] = xw_ref[...] * dis.reshape(N, 1)


def _scale(xw, degp):
    return pl.pallas_call(
        _scale_body,
        out_shape=jax.ShapeDtypeStruct((N, D), jnp.float32),
    )(xw, degp)


def _final_body(acc_ref, y_ref, degp_ref, b_ref, o_ref):
    deg = degp_ref[0] + degp_ref[1] + 1.0
    dis = jnp.where(deg > 0, lax.rsqrt(deg), 0.0)
    agg = acc_ref[0] + acc_ref[1] + y_ref[...]
    out = dis.reshape(N, 1) * agg + b_ref[...]
    m = jnp.max(out, axis=0, keepdims=True)
    z = jnp.exp(out - m)
    lse = jnp.log(jnp.sum(z, axis=0, keepdims=True))
    o_ref[...] = out - m - lse


def _final(acc, y, degp, b):
    return pl.pallas_call(
        _final_body,
        out_shape=jax.ShapeDtypeStruct((N, D), jnp.float32),
    )(acc, y, degp, b)


def kernel(x, edge_index, edge_weight, W, b):
    src = edge_index[0]
    dst = edge_index[1]
    pad = E2 - E
    src2 = jnp.pad(src, (0, pad)).reshape(NCH, K)
    dst2 = jnp.pad(dst, (0, pad)).reshape(NCH, K)
    w2 = jnp.pad(edge_weight, (0, pad)).reshape(NCH, K)
    degp = _deg_kernel(w2, dst2).reshape(NC, N)
    xw = _mm(x, W)
    y = _scale(xw, degp)
    acc = _msg_kernel(y, src2, dst2, w2)
    return _final(acc, y, degp, b)


# speedup vs baseline: 1.5560x; 1.0078x over previous
import dataclasses
import functools

import jax
import jax.numpy as jnp
from jax import lax
from jax.experimental import pallas as pl
from jax.experimental.pallas import tpu as pltpu
from jax.experimental.pallas import tpu_sc as plsc

N = 10000
E = 320000
D = 128
NC = 2
NS = 16
NW = NC * NS
K = 128
NCH = 2560
E2 = NCH * K
NRT = 80
HALF = 40
CFAST = 128
CSLOW = 32
PIECES_FAST = (40, 40, 40, 8)
PIECES_SLOW = (32,)
RB = 624
REXTRA = N - NS * RB
ZCHUNKS = ((0, 128), (128, 128), (256, 128), (384, 128), (512, 112))

_sc_params = pltpu.CompilerParams()
if "needs_layout_passes" in pltpu.CompilerParams.__dataclass_fields__:
    _sc_params = dataclasses.replace(_sc_params, needs_layout_passes=False)

_mesh = plsc.VectorSubcoreMesh(
    core_axis_name="c", subcore_axis_name="s", num_cores=NC, num_subcores=NS
)


def _deg_body(w2_hbm, dst2_hbm, out_hbm, deg_sh, w_all, didx_all, zbuf_v,
              xbuf_v, ssem):
    cid = lax.axis_index("c")
    sid = lax.axis_index("s")
    wid = cid * NS + sid

    @pl.loop(0, RB // 16)
    def _zero_buf(i):
        zbuf_v[pl.ds(i * 16, 16)] = jnp.zeros((16,), jnp.float32)

    pltpu.sync_copy(zbuf_v, deg_sh.at[pl.ds(sid * RB, RB)])

    @pl.when(sid == NS - 1)
    def _zero_extra():
        pltpu.sync_copy(zbuf_v.at[pl.ds(0, REXTRA)],
                        deg_sh.at[pl.ds(NS * RB, REXTRA)])

    pltpu.sync_copy(w2_hbm.at[pl.ds(wid * NRT, NRT)], w_all)
    pltpu.sync_copy(dst2_hbm.at[pl.ds(wid * NRT, NRT)], didx_all)

    plsc.subcore_barrier()

    @pl.loop(0, NRT)
    def _fire(i):
        pltpu.async_copy(w_all.at[i], deg_sh.at[didx_all.at[i]], ssem,
                         add=True)

    @pl.loop(0, NRT)
    def _drain(i):
        pltpu.make_async_copy(w_all.at[0], deg_sh.at[didx_all.at[0]],
                              ssem).wait()

    plsc.subcore_barrier()
    pltpu.sync_copy(deg_sh.at[pl.ds(sid * RB, RB)], zbuf_v)
    pltpu.sync_copy(zbuf_v, out_hbm.at[pl.ds(cid * N + sid * RB, RB)])

    @pl.when(sid == NS - 1)
    def _copy_extra():
        pltpu.sync_copy(deg_sh.at[pl.ds(NS * RB, REXTRA)], xbuf_v)
        pltpu.sync_copy(xbuf_v, out_hbm.at[pl.ds(cid * N + NS * RB, REXTRA)])


_deg_kernel = functools.partial(
    pl.kernel,
    out_type=jax.ShapeDtypeStruct((NC * N,), jnp.float32),
    mesh=_mesh,
    scratch_types=[
        pltpu.VMEM_SHARED((N,), jnp.float32),
        pltpu.VMEM((NRT, K), jnp.float32),
        pltpu.VMEM((NRT, K), jnp.int32),
        pltpu.VMEM((RB,), jnp.float32),
        pltpu.VMEM((REXTRA,), jnp.float32),
        pltpu.SemaphoreType.DMA,
    ],
)(_deg_body)


def _msg_body(y_hbm, src2_hbm, dst2_hbm, w2_hbm, out_hbm, acc_sh, r0, r1,
              src_h, dst_h, w_h, g0, g1, s0, s1):
    cid = lax.axis_index("c")
    sid = lax.axis_index("s")
    rows = (r0, r1)
    gsem = (g0, g1)
    ssem = (s0, s1)
    def issue_gather(i, b):
        pltpu.async_copy(y_hbm.at[src_h.at[i]], rows[b], gsem[b])

    def wait_gather(b):
        pltpu.make_async_copy(y_hbm.at[src_h.at[0]], rows[b], gsem[b]).wait()

    def issue_scatter(i, b):
        pltpu.async_copy(rows[b], acc_sh.at[dst_h.at[i]], ssem[b], add=True)

    def wait_scatter(b):
        pltpu.make_async_copy(rows[0], acc_sh.at[dst_h.at[0]],
                              ssem[b]).wait()

    @pl.loop(0, K)
    def _zero_buf(i):
        for j in range(D // 16):
            r0[i, pl.ds(j * 16, 16)] = jnp.zeros((16,), jnp.float32)

    for zoff, zsz in ZCHUNKS:
        pltpu.sync_copy(r0.at[pl.ds(0, zsz)],
                        acc_sh.at[pl.ds(sid * RB + zoff, zsz)])

    @pl.when(sid == NS - 1)
    def _zero_extra():
        pltpu.sync_copy(r0.at[pl.ds(0, REXTRA)],
                        acc_sh.at[pl.ds(NS * RB, REXTRA)])

    plsc.subcore_barrier()

    def emit_piece(base_row, nch):
        pltpu.sync_copy(src2_hbm.at[pl.ds(base_row, nch)],
                        src_h.at[pl.ds(0, nch)])
        pltpu.sync_copy(dst2_hbm.at[pl.ds(base_row, nch)],
                        dst_h.at[pl.ds(0, nch)])
        pltpu.sync_copy(w2_hbm.at[pl.ds(base_row, nch)],
                        w_h.at[pl.ds(0, nch)])

        issue_gather(0, 0)

        @pl.loop(0, nch, step=2)
        def _group(g):
            for k in range(2):
                i = g + k
                b = k
                bo = 1 - k

                @pl.when(i >= 1)
                def _drain_prev():
                    wait_scatter(bo)

                @pl.when(i + 1 < nch)
                def _prefetch():
                    issue_gather(i + 1, bo)

                wait_gather(b)

                @pl.loop(0, K, unroll=4)
                def _scale(e):
                    i16 = jnp.full((16,), i, jnp.int32)
                    e16 = jnp.full((16,), e, jnp.int32)
                    we = plsc.load_gather(w_h, [i16, e16])
                    for j in range(D // 16):
                        sl = (e, pl.ds(j * 16, 16))
                        rows[b][sl] = rows[b][sl] * we

                issue_scatter(i, b)

        wait_scatter(1)

    @pl.when(cid == 0)
    def _fast_core():
        base = sid * CFAST
        for p in PIECES_FAST:
            emit_piece(base, p)
            base += p

    @pl.when(cid != 0)
    def _slow_core():
        base = NS * CFAST + sid * CSLOW
        for p in PIECES_SLOW:
            emit_piece(base, p)
            base += p

    plsc.subcore_barrier()
    pltpu.sync_copy(acc_sh.at[pl.ds(sid * RB, RB)],
                    out_hbm.at[cid, pl.ds(sid * RB, RB)])

    @pl.when(sid == NS - 1)
    def _copy_extra():
        pltpu.sync_copy(acc_sh.at[pl.ds(NS * RB, REXTRA)],
                        out_hbm.at[cid, pl.ds(NS * RB, REXTRA)])


_msg_kernel = functools.partial(
    pl.kernel,
    out_type=jax.ShapeDtypeStruct((NC, N, D), jnp.float32),
    mesh=_mesh,
    scratch_types=[
        pltpu.VMEM_SHARED((N, D), jnp.float32),
        pltpu.VMEM((K, D), jnp.float32),
        pltpu.VMEM((K, D), jnp.float32),
        pltpu.VMEM((HALF, K), jnp.int32),
        pltpu.VMEM((HALF, K), jnp.int32),
        pltpu.VMEM((HALF, K), jnp.float32),
        pltpu.SemaphoreType.DMA,
        pltpu.SemaphoreType.DMA,
        pltpu.SemaphoreType.DMA,
        pltpu.SemaphoreType.DMA,
    ],
    compiler_params=_sc_params,
)(_msg_body)


def _mm_body(x_ref, w_ref, xw_ref):
    xw_ref[...] = jnp.dot(x_ref[...], w_ref[...],
                          preferred_element_type=jnp.float32)


def _mm(x, W):
    return pl.pallas_call(
        _mm_body,
        out_shape=jax.ShapeDtypeStruct((N, D), jnp.float32),
    )(x, W)


def _scale_body(xw_ref, degp_ref, y_ref):
    deg = degp_ref[0] + degp_ref[1] + 1.0
    dis = jnp.where(deg > 0, lax.rsqrt(deg), 0.0)
    y_ref[...] = xw_ref[...] * dis.reshape(N, 1)


def _scale(xw, degp):
    return pl.pallas_call(
        _scale_body,
        out_shape=jax.ShapeDtypeStruct((N, D), jnp.float32),
    )(xw, degp)


def _final_body(acc_ref, y_ref, degp_ref, b_ref, o_ref):
    deg = degp_ref[0] + degp_ref[1] + 1.0
    dis = jnp.where(deg > 0, lax.rsqrt(deg), 0.0)
    agg = acc_ref[0] + acc_ref[1] + y_ref[...]
    out = dis.reshape(N, 1) * agg + b_ref[...]
    m = jnp.max(out, axis=0, keepdims=True)
    z = jnp.exp(out - m)
    lse = jnp.log(jnp.sum(z, axis=0, keepdims=True))
    o_ref[...] = out - m - lse


def _final(acc, y, degp, b):
    return pl.pallas_call(
        _final_body,
        out_shape=jax.ShapeDtypeStruct((N, D), jnp.float32),
    )(acc, y, degp, b)


def kernel(x, edge_index, edge_weight, W, b):
    src = edge_index[0]
    dst = edge_index[1]
    pad = E2 - E
    src2 = jnp.pad(src, (0, pad)).reshape(NCH, K)
    dst2 = jnp.pad(dst, (0, pad)).reshape(NCH, K)
    w2 = jnp.pad(edge_weight, (0, pad)).reshape(NCH, K)
    degp = _deg_kernel(w2, dst2).reshape(NC, N)
    xw = _mm(x, W)
    y = _scale(xw, degp)
    acc = _msg_kernel(y, src2, dst2, w2)
    return _final(acc, y, degp, b)


# speedup vs baseline: 1.5697x; 1.0088x over previous
import dataclasses
import functools

import jax
import jax.numpy as jnp
from jax import lax
from jax.experimental import pallas as pl
from jax.experimental.pallas import tpu as pltpu
from jax.experimental.pallas import tpu_sc as plsc

N = 10000
E = 320000
D = 128
NC = 2
NS = 16
NW = NC * NS
K = 128
NCH = 2560
E2 = NCH * K
NRT = 80
HALF = 40
CFAST = 136
CSLOW = 24
PIECES_FAST = (40, 40, 40, 16)
PIECES_SLOW = (24,)
RB = 624
REXTRA = N - NS * RB
ZCHUNKS = ((0, 128), (128, 128), (256, 128), (384, 128), (512, 112))

_sc_params = pltpu.CompilerParams()
if "needs_layout_passes" in pltpu.CompilerParams.__dataclass_fields__:
    _sc_params = dataclasses.replace(_sc_params, needs_layout_passes=False)

_mesh = plsc.VectorSubcoreMesh(
    core_axis_name="c", subcore_axis_name="s", num_cores=NC, num_subcores=NS
)


def _deg_body(w2_hbm, dst2_hbm, out_hbm, deg_sh, w_all, didx_all, zbuf_v,
              xbuf_v, ssem):
    cid = lax.axis_index("c")
    sid = lax.axis_index("s")
    wid = cid * NS + sid

    @pl.loop(0, RB // 16)
    def _zero_buf(i):
        zbuf_v[pl.ds(i * 16, 16)] = jnp.zeros((16,), jnp.float32)

    pltpu.sync_copy(zbuf_v, deg_sh.at[pl.ds(sid * RB, RB)])

    @pl.when(sid == NS - 1)
    def _zero_extra():
        pltpu.sync_copy(zbuf_v.at[pl.ds(0, REXTRA)],
                        deg_sh.at[pl.ds(NS * RB, REXTRA)])

    pltpu.sync_copy(w2_hbm.at[pl.ds(wid * NRT, NRT)], w_all)
    pltpu.sync_copy(dst2_hbm.at[pl.ds(wid * NRT, NRT)], didx_all)

    plsc.subcore_barrier()

    @pl.loop(0, NRT)
    def _fire(i):
        pltpu.async_copy(w_all.at[i], deg_sh.at[didx_all.at[i]], ssem,
                         add=True)

    @pl.loop(0, NRT)
    def _drain(i):
        pltpu.make_async_copy(w_all.at[0], deg_sh.at[didx_all.at[0]],
                              ssem).wait()

    plsc.subcore_barrier()
    pltpu.sync_copy(deg_sh.at[pl.ds(sid * RB, RB)], zbuf_v)
    pltpu.sync_copy(zbuf_v, out_hbm.at[pl.ds(cid * N + sid * RB, RB)])

    @pl.when(sid == NS - 1)
    def _copy_extra():
        pltpu.sync_copy(deg_sh.at[pl.ds(NS * RB, REXTRA)], xbuf_v)
        pltpu.sync_copy(xbuf_v, out_hbm.at[pl.ds(cid * N + NS * RB, REXTRA)])


_deg_kernel = functools.partial(
    pl.kernel,
    out_type=jax.ShapeDtypeStruct((NC * N,), jnp.float32),
    mesh=_mesh,
    scratch_types=[
        pltpu.VMEM_SHARED((N,), jnp.float32),
        pltpu.VMEM((NRT, K), jnp.float32),
        pltpu.VMEM((NRT, K), jnp.int32),
        pltpu.VMEM((RB,), jnp.float32),
        pltpu.VMEM((REXTRA,), jnp.float32),
        pltpu.SemaphoreType.DMA,
    ],
)(_deg_body)


def _msg_body(y_hbm, src2_hbm, dst2_hbm, w2_hbm, out_hbm, acc_sh, r0, r1,
              src_h, dst_h, w_h, g0, g1, s0, s1):
    cid = lax.axis_index("c")
    sid = lax.axis_index("s")
    rows = (r0, r1)
    gsem = (g0, g1)
    ssem = (s0, s1)
    def issue_gather(i, b):
        pltpu.async_copy(y_hbm.at[src_h.at[i]], rows[b], gsem[b])

    def wait_gather(b):
        pltpu.make_async_copy(y_hbm.at[src_h.at[0]], rows[b], gsem[b]).wait()

    def issue_scatter(i, b):
        pltpu.async_copy(rows[b], acc_sh.at[dst_h.at[i]], ssem[b], add=True)

    def wait_scatter(b):
        pltpu.make_async_copy(rows[0], acc_sh.at[dst_h.at[0]],
                              ssem[b]).wait()

    @pl.loop(0, K)
    def _zero_buf(i):
        for j in range(D // 16):
            r0[i, pl.ds(j * 16, 16)] = jnp.zeros((16,), jnp.float32)

    for zoff, zsz in ZCHUNKS:
        pltpu.sync_copy(r0.at[pl.ds(0, zsz)],
                        acc_sh.at[pl.ds(sid * RB + zoff, zsz)])

    @pl.when(sid == NS - 1)
    def _zero_extra():
        pltpu.sync_copy(r0.at[pl.ds(0, REXTRA)],
                        acc_sh.at[pl.ds(NS * RB, REXTRA)])

    plsc.subcore_barrier()

    def emit_piece(base_row, nch):
        pltpu.sync_copy(src2_hbm.at[pl.ds(base_row, nch)],
                        src_h.at[pl.ds(0, nch)])
        pltpu.sync_copy(dst2_hbm.at[pl.ds(base_row, nch)],
                        dst_h.at[pl.ds(0, nch)])
        pltpu.sync_copy(w2_hbm.at[pl.ds(base_row, nch)],
                        w_h.at[pl.ds(0, nch)])

        issue_gather(0, 0)

        @pl.loop(0, nch, step=2)
        def _group(g):
            for k in range(2):
                i = g + k
                b = k
                bo = 1 - k

                @pl.when(i >= 1)
                def _drain_prev():
                    wait_scatter(bo)

                @pl.when(i + 1 < nch)
                def _prefetch():
                    issue_gather(i + 1, bo)

                wait_gather(b)

                @pl.loop(0, K, unroll=4)
                def _scale(e):
                    i16 = jnp.full((16,), i, jnp.int32)
                    e16 = jnp.full((16,), e, jnp.int32)
                    we = plsc.load_gather(w_h, [i16, e16])
                    for j in range(D // 16):
                        sl = (e, pl.ds(j * 16, 16))
                        rows[b][sl] = rows[b][sl] * we

                issue_scatter(i, b)

        wait_scatter(1)

    @pl.when(cid == 0)
    def _fast_core():
        base = sid * CFAST
        for p in PIECES_FAST:
            emit_piece(base, p)
            base += p

    @pl.when(cid != 0)
    def _slow_core():
        base = NS * CFAST + sid * CSLOW
        for p in PIECES_SLOW:
            emit_piece(base, p)
            base += p

    plsc.subcore_barrier()
    pltpu.sync_copy(acc_sh.at[pl.ds(sid * RB, RB)],
                    out_hbm.at[cid, pl.ds(sid * RB, RB)])

    @pl.when(sid == NS - 1)
    def _copy_extra():
        pltpu.sync_copy(acc_sh.at[pl.ds(NS * RB, REXTRA)],
                        out_hbm.at[cid, pl.ds(NS * RB, REXTRA)])


_msg_kernel = functools.partial(
    pl.kernel,
    out_type=jax.ShapeDtypeStruct((NC, N, D), jnp.float32),
    mesh=_mesh,
    scratch_types=[
        pltpu.VMEM_SHARED((N, D), jnp.float32),
        pltpu.VMEM((K, D), jnp.float32),
        pltpu.VMEM((K, D), jnp.float32),
        pltpu.VMEM((HALF, K), jnp.int32),
        pltpu.VMEM((HALF, K), jnp.int32),
        pltpu.VMEM((HALF, K), jnp.float32),
        pltpu.SemaphoreType.DMA,
        pltpu.SemaphoreType.DMA,
        pltpu.SemaphoreType.DMA,
        pltpu.SemaphoreType.DMA,
    ],
    compiler_params=_sc_params,
)(_msg_body)


def _mm_body(x_ref, w_ref, xw_ref):
    xw_ref[...] = jnp.dot(x_ref[...], w_ref[...],
                          preferred_element_type=jnp.float32)


def _mm(x, W):
    return pl.pallas_call(
        _mm_body,
        out_shape=jax.ShapeDtypeStruct((N, D), jnp.float32),
    )(x, W)


def _scale_body(xw_ref, degp_ref, y_ref):
    deg = degp_ref[0] + degp_ref[1] + 1.0
    dis = jnp.where(deg > 0, lax.rsqrt(deg), 0.0)
    y_ref[...] = xw_ref[...] * dis.reshape(N, 1)


def _scale(xw, degp):
    return pl.pallas_call(
        _scale_body,
        out_shape=jax.ShapeDtypeStruct((N, D), jnp.float32),
    )(xw, degp)


def _final_body(acc_ref, y_ref, degp_ref, b_ref, o_ref):
    deg = degp_ref[0] + degp_ref[1] + 1.0
    dis = jnp.where(deg > 0, lax.rsqrt(deg), 0.0)
    agg = acc_ref[0] + acc_ref[1] + y_ref[...]
    out = dis.reshape(N, 1) * agg + b_ref[...]
    m = jnp.max(out, axis=0, keepdims=True)
    z = jnp.exp(out - m)
    lse = jnp.log(jnp.sum(z, axis=0, keepdims=True))
    o_ref[...] = out - m - lse


def _final(acc, y, degp, b):
    return pl.pallas_call(
        _final_body,
        out_shape=jax.ShapeDtypeStruct((N, D), jnp.float32),
    )(acc, y, degp, b)


def kernel(x, edge_index, edge_weight, W, b):
    src = edge_index[0]
    dst = edge_index[1]
    pad = E2 - E
    src2 = jnp.pad(src, (0, pad)).reshape(NCH, K)
    dst2 = jnp.pad(dst, (0, pad)).reshape(NCH, K)
    w2 = jnp.pad(edge_weight, (0, pad)).reshape(NCH, K)
    degp = _deg_kernel(w2, dst2).reshape(NC, N)
    xw = _mm(x, W)
    y = _scale(xw, degp)
    acc = _msg_kernel(y, src2, dst2, w2)
    return _final(acc, y, degp, b)


# speedup vs baseline: 1.6057x; 1.0230x over previous
import dataclasses
import functools

import jax
import jax.numpy as jnp
from jax import lax
from jax.experimental import pallas as pl
from jax.experimental.pallas import tpu as pltpu
from jax.experimental.pallas import tpu_sc as plsc

N = 10000
E = 320000
D = 128
NC = 2
NS = 16
NW = NC * NS
K = 128
NCH = 2560
E2 = NCH * K
NRT = 80
HALF = 40
CFAST = 144
CSLOW = 16
PIECES_FAST = (40, 40, 40, 24)
PIECES_SLOW = (16,)
RB = 624
REXTRA = N - NS * RB
ZCHUNKS = ((0, 128), (128, 128), (256, 128), (384, 128), (512, 112))

_sc_params = pltpu.CompilerParams()
if "needs_layout_passes" in pltpu.CompilerParams.__dataclass_fields__:
    _sc_params = dataclasses.replace(_sc_params, needs_layout_passes=False)

_mesh = plsc.VectorSubcoreMesh(
    core_axis_name="c", subcore_axis_name="s", num_cores=NC, num_subcores=NS
)


def _deg_body(w2_hbm, dst2_hbm, out_hbm, deg_sh, w_all, didx_all, zbuf_v,
              xbuf_v, ssem):
    cid = lax.axis_index("c")
    sid = lax.axis_index("s")
    wid = cid * NS + sid

    @pl.loop(0, RB // 16)
    def _zero_buf(i):
        zbuf_v[pl.ds(i * 16, 16)] = jnp.zeros((16,), jnp.float32)

    pltpu.sync_copy(zbuf_v, deg_sh.at[pl.ds(sid * RB, RB)])

    @pl.when(sid == NS - 1)
    def _zero_extra():
        pltpu.sync_copy(zbuf_v.at[pl.ds(0, REXTRA)],
                        deg_sh.at[pl.ds(NS * RB, REXTRA)])

    pltpu.sync_copy(w2_hbm.at[pl.ds(wid * NRT, NRT)], w_all)
    pltpu.sync_copy(dst2_hbm.at[pl.ds(wid * NRT, NRT)], didx_all)

    plsc.subcore_barrier()

    @pl.loop(0, NRT)
    def _fire(i):
        pltpu.async_copy(w_all.at[i], deg_sh.at[didx_all.at[i]], ssem,
                         add=True)

    @pl.loop(0, NRT)
    def _drain(i):
        pltpu.make_async_copy(w_all.at[0], deg_sh.at[didx_all.at[0]],
                              ssem).wait()

    plsc.subcore_barrier()
    pltpu.sync_copy(deg_sh.at[pl.ds(sid * RB, RB)], zbuf_v)
    pltpu.sync_copy(zbuf_v, out_hbm.at[pl.ds(cid * N + sid * RB, RB)])

    @pl.when(sid == NS - 1)
    def _copy_extra():
        pltpu.sync_copy(deg_sh.at[pl.ds(NS * RB, REXTRA)], xbuf_v)
        pltpu.sync_copy(xbuf_v, out_hbm.at[pl.ds(cid * N + NS * RB, REXTRA)])


_deg_kernel = functools.partial(
    pl.kernel,
    out_type=jax.ShapeDtypeStruct((NC * N,), jnp.float32),
    mesh=_mesh,
    scratch_types=[
        pltpu.VMEM_SHARED((N,), jnp.float32),
        pltpu.VMEM((NRT, K), jnp.float32),
        pltpu.VMEM((NRT, K), jnp.int32),
        pltpu.VMEM((RB,), jnp.float32),
        pltpu.VMEM((REXTRA,), jnp.float32),
        pltpu.SemaphoreType.DMA,
    ],
)(_deg_body)


def _msg_body(y_hbm, src2_hbm, dst2_hbm, w2_hbm, out_hbm, acc_sh, r0, r1,
              src_h, dst_h, w_h, g0, g1, s0, s1):
    cid = lax.axis_index("c")
    sid = lax.axis_index("s")
    rows = (r0, r1)
    gsem = (g0, g1)
    ssem = (s0, s1)
    def issue_gather(i, b):
        pltpu.async_copy(y_hbm.at[src_h.at[i]], rows[b], gsem[b])

    def wait_gather(b):
        pltpu.make_async_copy(y_hbm.at[src_h.at[0]], rows[b], gsem[b]).wait()

    def issue_scatter(i, b):
        pltpu.async_copy(rows[b], acc_sh.at[dst_h.at[i]], ssem[b], add=True)

    def wait_scatter(b):
        pltpu.make_async_copy(rows[0], acc_sh.at[dst_h.at[0]],
                              ssem[b]).wait()

    @pl.loop(0, K)
    def _zero_buf(i):
        for j in range(D // 16):
            r0[i, pl.ds(j * 16, 16)] = jnp.zeros((16,), jnp.float32)

    for zoff, zsz in ZCHUNKS:
        pltpu.sync_copy(r0.at[pl.ds(0, zsz)],
                        acc_sh.at[pl.ds(sid * RB + zoff, zsz)])

    @pl.when(sid == NS - 1)
    def _zero_extra():
        pltpu.sync_copy(r0.at[pl.ds(0, REXTRA)],
                        acc_sh.at[pl.ds(NS * RB, REXTRA)])

    plsc.subcore_barrier()

    def emit_piece(base_row, nch):
        pltpu.sync_copy(src2_hbm.at[pl.ds(base_row, nch)],
                        src_h.at[pl.ds(0, nch)])
        pltpu.sync_copy(dst2_hbm.at[pl.ds(base_row, nch)],
                        dst_h.at[pl.ds(0, nch)])
        pltpu.sync_copy(w2_hbm.at[pl.ds(base_row, nch)],
                        w_h.at[pl.ds(0, nch)])

        issue_gather(0, 0)

        @pl.loop(0, nch, step=2)
        def _group(g):
            for k in range(2):
                i = g + k
                b = k
                bo = 1 - k

                @pl.when(i >= 1)
                def _drain_prev():
                    wait_scatter(bo)

                @pl.when(i + 1 < nch)
                def _prefetch():
                    issue_gather(i + 1, bo)

                wait_gather(b)

                @pl.loop(0, K, unroll=4)
                def _scale(e):
                    i16 = jnp.full((16,), i, jnp.int32)
                    e16 = jnp.full((16,), e, jnp.int32)
                    we = plsc.load_gather(w_h, [i16, e16])
                    for j in range(D // 16):
                        sl = (e, pl.ds(j * 16, 16))
                        rows[b][sl] = rows[b][sl] * we

                issue_scatter(i, b)

        wait_scatter(1)

    @pl.when(cid == 0)
    def _fast_core():
        base = sid * CFAST
        for p in PIECES_FAST:
            emit_piece(base, p)
            base += p

    @pl.when(cid != 0)
    def _slow_core():
        base = NS * CFAST + sid * CSLOW
        for p in PIECES_SLOW:
            emit_piece(base, p)
            base += p

    plsc.subcore_barrier()
    pltpu.sync_copy(acc_sh.at[pl.ds(sid * RB, RB)],
                    out_hbm.at[cid, pl.ds(sid * RB, RB)])

    @pl.when(sid == NS - 1)
    def _copy_extra():
        pltpu.sync_copy(acc_sh.at[pl.ds(NS * RB, REXTRA)],
                        out_hbm.at[cid, pl.ds(NS * RB, REXTRA)])


_msg_kernel = functools.partial(
    pl.kernel,
    out_type=jax.ShapeDtypeStruct((NC, N, D), jnp.float32),
    mesh=_mesh,
    scratch_types=[
        pltpu.VMEM_SHARED((N, D), jnp.float32),
        pltpu.VMEM((K, D), jnp.float32),
        pltpu.VMEM((K, D), jnp.float32),
        pltpu.VMEM((HALF, K), jnp.int32),
        pltpu.VMEM((HALF, K), jnp.int32),
        pltpu.VMEM((HALF, K), jnp.float32),
        pltpu.SemaphoreType.DMA,
        pltpu.SemaphoreType.DMA,
        pltpu.SemaphoreType.DMA,
        pltpu.SemaphoreType.DMA,
    ],
    compiler_params=_sc_params,
)(_msg_body)


def _mm_body(x_ref, w_ref, xw_ref):
    xw_ref[...] = jnp.dot(x_ref[...], w_ref[...],
                          preferred_element_type=jnp.float32)


def _mm(x, W):
    return pl.pallas_call(
        _mm_body,
        out_shape=jax.ShapeDtypeStruct((N, D), jnp.float32),
    )(x, W)


def _scale_body(xw_ref, degp_ref, y_ref):
    deg = degp_ref[0] + degp_ref[1] + 1.0
    dis = jnp.where(deg > 0, lax.rsqrt(deg), 0.0)
    y_ref[...] = xw_ref[...] * dis.reshape(N, 1)


def _scale(xw, degp):
    return pl.pallas_call(
        _scale_body,
        out_shape=jax.ShapeDtypeStruct((N, D), jnp.float32),
    )(xw, degp)


def _final_body(acc_ref, y_ref, degp_ref, b_ref, o_ref):
    deg = degp_ref[0] + degp_ref[1] + 1.0
    dis = jnp.where(deg > 0, lax.rsqrt(deg), 0.0)
    agg = acc_ref[0] + acc_ref[1] + y_ref[...]
    out = dis.reshape(N, 1) * agg + b_ref[...]
    m = jnp.max(out, axis=0, keepdims=True)
    z = jnp.exp(out - m)
    lse = jnp.log(jnp.sum(z, axis=0, keepdims=True))
    o_ref[...] = out - m - lse


def _final(acc, y, degp, b):
    return pl.pallas_call(
        _final_body,
        out_shape=jax.ShapeDtypeStruct((N, D), jnp.float32),
    )(acc, y, degp, b)


def kernel(x, edge_index, edge_weight, W, b):
    src = edge_index[0]
    dst = edge_index[1]
    pad = E2 - E
    src2 = jnp.pad(src, (0, pad)).reshape(NCH, K)
    dst2 = jnp.pad(dst, (0, pad)).reshape(NCH, K)
    w2 = jnp.pad(edge_weight, (0, pad)).reshape(NCH, K)
    degp = _deg_kernel(w2, dst2).reshape(NC, N)
    xw = _mm(x, W)
    y = _scale(xw, degp)
    acc = _msg_kernel(y, src2, dst2, w2)
    return _final(acc, y, degp, b)
